# Initial kernel scaffold; baseline (speedup 1.0000x reference)
import jax
import jax.numpy as jnp
from jax.experimental import pallas as pl


def kernel(x, edge_index, W, b):
    raise NotImplementedError("write your pallas kernel here")


# speedup vs baseline: 25.6779x; 25.6779x over previous
import functools

import jax
import jax.numpy as jnp
from jax import lax
from jax.experimental import pallas as pl
from jax.experimental.pallas import tpu as pltpu
from jax.experimental.pallas import tpu_sc as plsc

N = 10000
E = 320000
D = 128

NC = 2
NS = 16
NW = NC * NS
E_TILE = E // NW
K = 80
CHUNKS = E_TILE // K
N_PAD = 10240
ROWS_PER_SUB = N_PAD // NS

_MESH = plsc.VectorSubcoreMesh(
    core_axis_name="c", subcore_axis_name="s", num_cores=NC, num_subcores=NS
)


@functools.partial(
    pl.kernel,
    out_type=jax.ShapeDtypeStruct((NC, N_PAD, 16), jnp.float32),
    mesh=_MESH,
    scratch_types=[
        pltpu.VMEM((CHUNKS, K), jnp.int32),
        pltpu.VMEM((K, 16), jnp.float32),
        pltpu.VMEM_SHARED((N_PAD, 16), jnp.float32),
    ],
)
def _sc_degree(dst_hbm, zeros_hbm, out_hbm, idx_v, ones_v, acc):
    cid = lax.axis_index("c")
    sid = lax.axis_index("s")
    wid = sid * NC + cid

    pltpu.sync_copy(
        zeros_hbm.at[pl.ds(sid * ROWS_PER_SUB, ROWS_PER_SUB)],
        acc.at[pl.ds(sid * ROWS_PER_SUB, ROWS_PER_SUB)],
    )
    @pl.loop(0, K)
    def _(j):
        ones_v[j, :] = jnp.full((16,), 1.0, jnp.float32)

    pltpu.sync_copy(dst_hbm.at[wid], idx_v)
    plsc.subcore_barrier()

    @pl.loop(0, CHUNKS)
    def _(j):
        pltpu.sync_copy(ones_v, acc.at[idx_v.at[j]], add=True)

    plsc.subcore_barrier()
    pltpu.sync_copy(
        acc.at[pl.ds(sid * ROWS_PER_SUB, ROWS_PER_SUB)],
        out_hbm.at[cid].at[pl.ds(sid * ROWS_PER_SUB, ROWS_PER_SUB)],
    )


@functools.partial(
    pl.kernel,
    out_type=jax.ShapeDtypeStruct((NC, N_PAD, D), jnp.float32),
    mesh=_MESH,
    scratch_types=[
        pltpu.VMEM((CHUNKS, K), jnp.int32),
        pltpu.VMEM((CHUNKS, K), jnp.int32),
        pltpu.VMEM((K, D), jnp.float32),
        pltpu.VMEM_SHARED((N_PAD, D), jnp.float32),
        pltpu.SemaphoreType.DMA,
    ],
)
def _sc_gather_scatter(g_hbm, src_hbm, dst_hbm, zeros_hbm, out_hbm,
                       src_v, dst_v, rows_v, acc, sem):
    cid = lax.axis_index("c")
    sid = lax.axis_index("s")
    wid = sid * NC + cid

    pltpu.sync_copy(
        zeros_hbm.at[pl.ds(sid * ROWS_PER_SUB, ROWS_PER_SUB)],
        acc.at[pl.ds(sid * ROWS_PER_SUB, ROWS_PER_SUB)],
    )
    pltpu.sync_copy(src_hbm.at[wid], src_v)
    pltpu.sync_copy(dst_hbm.at[wid], dst_v)
    plsc.subcore_barrier()

    @pl.loop(0, CHUNKS)
    def _(j):
        pltpu.async_copy(g_hbm.at[src_v.at[j]], rows_v, sem).wait()
        pltpu.sync_copy(rows_v, acc.at[dst_v.at[j]], add=True)

    plsc.subcore_barrier()
    pltpu.sync_copy(
        acc.at[pl.ds(sid * ROWS_PER_SUB, ROWS_PER_SUB)],
        out_hbm.at[cid].at[pl.ds(sid * ROWS_PER_SUB, ROWS_PER_SUB)],
    )


_RB = 2000
_GRID = N // _RB


def _matmul_body(x_ref, w_ref, h_ref):
    h_ref[...] = jnp.dot(x_ref[...], w_ref[...],
                         preferred_element_type=jnp.float32)


def _tc_matmul(x, W):
    return pl.pallas_call(
        _matmul_body,
        grid=(_GRID,),
        in_specs=[
            pl.BlockSpec((_RB, D), lambda i: (i, 0)),
            pl.BlockSpec((D, D), lambda i: (0, 0)),
        ],
        out_specs=pl.BlockSpec((_RB, D), lambda i: (i, 0)),
        out_shape=jax.ShapeDtypeStruct((N, D), jnp.float32),
    )(x, W)


def _dis_from_parts(dp):
    deg = dp[0, :, 0:1] + dp[1, :, 0:1] + 1.0
    return lax.rsqrt(deg)


def _scale_body(h_ref, dp_ref, g_ref):
    g_ref[...] = _dis_from_parts(dp_ref[...]) * h_ref[...]


def _tc_scale(h, deg_parts):
    return pl.pallas_call(
        _scale_body,
        grid=(_GRID,),
        in_specs=[
            pl.BlockSpec((_RB, D), lambda i: (i, 0)),
            pl.BlockSpec((NC, _RB, 16), lambda i: (0, i, 0)),
        ],
        out_specs=pl.BlockSpec((_RB, D), lambda i: (i, 0)),
        out_shape=jax.ShapeDtypeStruct((N, D), jnp.float32),
    )(h, deg_parts)


def _final_body(p0_ref, p1_ref, g_ref, dp_ref, b_ref, o_ref):
    dis = _dis_from_parts(dp_ref[...])
    o = dis * (p0_ref[...] + p1_ref[...] + g_ref[...]) + b_ref[...]
    m = jnp.max(o, axis=1, keepdims=True)
    e = jnp.exp(o - m)
    s = jnp.sum(e, axis=1, keepdims=True)
    o_ref[...] = (o - m) - jnp.log(s)


def _tc_final(p0, p1, g, deg_parts, b):
    return pl.pallas_call(
        _final_body,
        grid=(_GRID,),
        in_specs=[
            pl.BlockSpec((_RB, D), lambda i: (i, 0)),
            pl.BlockSpec((_RB, D), lambda i: (i, 0)),
            pl.BlockSpec((_RB, D), lambda i: (i, 0)),
            pl.BlockSpec((NC, _RB, 16), lambda i: (0, i, 0)),
            pl.BlockSpec((1, D), lambda i: (0, 0)),
        ],
        out_specs=pl.BlockSpec((_RB, D), lambda i: (i, 0)),
        out_shape=jax.ShapeDtypeStruct((N, D), jnp.float32),
    )(p0, p1, g, deg_parts, b)


@jax.jit
def kernel(x, edge_index, W, b):
    src3 = edge_index[0].reshape(NW, CHUNKS, K)
    dst3 = edge_index[1].reshape(NW, CHUNKS, K)
    zeros16 = jnp.zeros((N_PAD, 16), jnp.float32)
    zerosD = jnp.zeros((N_PAD, D), jnp.float32)

    deg_parts = _sc_degree(dst3, zeros16)
    h = _tc_matmul(x, W)
    g = _tc_scale(h, deg_parts)
    parts = _sc_gather_scatter(g, src3, dst3, zerosD)
    return _tc_final(parts[0], parts[1], g, deg_parts, b.reshape(1, D))


# speedup vs baseline: 36.5284x; 1.4226x over previous
import functools

import jax
import jax.numpy as jnp
from jax import lax
from jax.experimental import pallas as pl
from jax.experimental.pallas import tpu as pltpu
from jax.experimental.pallas import tpu_sc as plsc

N = 10000
E = 320000
D = 128

NC = 2
NS = 16
NW = NC * NS
N_PAD = 10240
ROWS_PER_SUB = N_PAD // NS

KD = 80
CHUNKSD = (E // NW) // KD

K = 100
CHUNKS = (E // NW) // K
HALVES = 2
CH_H = CHUNKS // HALVES
NBUF = 2
SB = 10
SBLOCKS_H = CH_H // SB

_MESH = plsc.VectorSubcoreMesh(
    core_axis_name="c", subcore_axis_name="s", num_cores=NC, num_subcores=NS
)


@functools.partial(
    pl.kernel,
    out_type=jax.ShapeDtypeStruct((NC, N_PAD, 16), jnp.float32),
    mesh=_MESH,
    scratch_types=[
        pltpu.VMEM((CHUNKSD, KD), jnp.int32),
        pltpu.VMEM((KD, 16), jnp.float32),
        pltpu.VMEM_SHARED((N_PAD, 16), jnp.float32),
        pltpu.SemaphoreType.DMA,
    ],
)
def _sc_degree(dst_hbm, zeros_hbm, out_hbm, idx_v, ones_v, acc, sem):
    cid = lax.axis_index("c")
    sid = lax.axis_index("s")
    wid = sid * NC + cid

    pltpu.sync_copy(
        zeros_hbm.at[pl.ds(sid * ROWS_PER_SUB, ROWS_PER_SUB)],
        acc.at[pl.ds(sid * ROWS_PER_SUB, ROWS_PER_SUB)],
    )
    @pl.loop(0, KD)
    def _(j):
        ones_v[j, :] = jnp.full((16,), 1.0, jnp.float32)

    pltpu.sync_copy(dst_hbm.at[wid], idx_v)
    plsc.subcore_barrier()

    @pl.loop(0, CHUNKSD // 5)
    def _(gr):
        descs = [
            pltpu.async_copy(ones_v, acc.at[idx_v.at[gr * 5 + t]], sem,
                             add=True)
            for t in range(5)
        ]
        for d in descs:
            d.wait()

    plsc.subcore_barrier()
    pltpu.sync_copy(
        acc.at[pl.ds(sid * ROWS_PER_SUB, ROWS_PER_SUB)],
        out_hbm.at[cid].at[pl.ds(sid * ROWS_PER_SUB, ROWS_PER_SUB)],
    )


@functools.partial(
    pl.kernel,
    out_type=jax.ShapeDtypeStruct((NC, N_PAD, D), jnp.float32),
    mesh=_MESH,
    scratch_types=[
        pltpu.VMEM((CH_H, K), jnp.int32),
        pltpu.VMEM((CH_H, K), jnp.int32),
        pltpu.VMEM((NBUF, K, D), jnp.float32),
        pltpu.VMEM_SHARED((N_PAD, D), jnp.float32),
        pltpu.SemaphoreType.DMA((NBUF,)),
        pltpu.SemaphoreType.DMA((NBUF,)),
    ],
)
def _sc_gather_scatter(g_hbm, src_hbm, dst_hbm, zeros_hbm, out_hbm,
                       src_v, dst_v, rows_v, acc, gsem, ssem):
    cid = lax.axis_index("c")
    sid = lax.axis_index("s")
    wid = sid * NC + cid

    pltpu.sync_copy(
        zeros_hbm.at[pl.ds(sid * ROWS_PER_SUB, ROWS_PER_SUB)],
        acc.at[pl.ds(sid * ROWS_PER_SUB, ROWS_PER_SUB)],
    )
    plsc.subcore_barrier()

    for h in range(HALVES):
        pltpu.sync_copy(src_hbm.at[wid].at[h], src_v)
        pltpu.sync_copy(dst_hbm.at[wid].at[h], dst_v)

        @pl.loop(0, SBLOCKS_H)
        def _(sb):
            base = sb * SB
            gds = [None] * NBUF
            sds = [None] * NBUF
            gds[0] = pltpu.async_copy(g_hbm.at[src_v.at[base]],
                                      rows_v.at[0], gsem.at[0])
            for i in range(SB):
                b = i % 2
                nb = 1 - b
                if i + 1 < SB:
                    if i >= 1:
                        sds[nb].wait()
                    gds[nb] = pltpu.async_copy(
                        g_hbm.at[src_v.at[base + i + 1]], rows_v.at[nb],
                        gsem.at[nb])
                gds[b].wait()
                sds[b] = pltpu.async_copy(rows_v.at[b],
                                          acc.at[dst_v.at[base + i]],
                                          ssem.at[b], add=True)
            sds[0].wait()
            sds[1].wait()

    plsc.subcore_barrier()
    pltpu.sync_copy(
        acc.at[pl.ds(sid * ROWS_PER_SUB, ROWS_PER_SUB)],
        out_hbm.at[cid].at[pl.ds(sid * ROWS_PER_SUB, ROWS_PER_SUB)],
    )


_RB = 2000
_GRID = N // _RB


def _matmul_body(x_ref, w_ref, h_ref):
    h_ref[...] = jnp.dot(x_ref[...], w_ref[...],
                         preferred_element_type=jnp.float32)


def _tc_matmul(x, W):
    return pl.pallas_call(
        _matmul_body,
        grid=(_GRID,),
        in_specs=[
            pl.BlockSpec((_RB, D), lambda i: (i, 0)),
            pl.BlockSpec((D, D), lambda i: (0, 0)),
        ],
        out_specs=pl.BlockSpec((_RB, D), lambda i: (i, 0)),
        out_shape=jax.ShapeDtypeStruct((N, D), jnp.float32),
    )(x, W)


def _dis_from_parts(dp):
    deg = dp[0, :, 0:1] + dp[1, :, 0:1] + 1.0
    return lax.rsqrt(deg)


def _scale_body(h_ref, dp_ref, g_ref):
    g_ref[...] = _dis_from_parts(dp_ref[...]) * h_ref[...]


def _tc_scale(h, deg_parts):
    return pl.pallas_call(
        _scale_body,
        grid=(_GRID,),
        in_specs=[
            pl.BlockSpec((_RB, D), lambda i: (i, 0)),
            pl.BlockSpec((NC, _RB, 16), lambda i: (0, i, 0)),
        ],
        out_specs=pl.BlockSpec((_RB, D), lambda i: (i, 0)),
        out_shape=jax.ShapeDtypeStruct((N, D), jnp.float32),
    )(h, deg_parts)


def _final_body(p0_ref, p1_ref, g_ref, dp_ref, b_ref, o_ref):
    dis = _dis_from_parts(dp_ref[...])
    o = dis * (p0_ref[...] + p1_ref[...] + g_ref[...]) + b_ref[...]
    m = jnp.max(o, axis=1, keepdims=True)
    e = jnp.exp(o - m)
    z = jnp.sum(e, axis=1, keepdims=True)
    o_ref[...] = (o - m) - jnp.log(z)


def _tc_final(p0, p1, g, deg_parts, b):
    return pl.pallas_call(
        _final_body,
        grid=(_GRID,),
        in_specs=[
            pl.BlockSpec((_RB, D), lambda i: (i, 0)),
            pl.BlockSpec((_RB, D), lambda i: (i, 0)),
            pl.BlockSpec((_RB, D), lambda i: (i, 0)),
            pl.BlockSpec((NC, _RB, 16), lambda i: (0, i, 0)),
            pl.BlockSpec((1, D), lambda i: (0, 0)),
        ],
        out_specs=pl.BlockSpec((_RB, D), lambda i: (i, 0)),
        out_shape=jax.ShapeDtypeStruct((N, D), jnp.float32),
    )(p0, p1, g, deg_parts, b)


@jax.jit
def kernel(x, edge_index, W, b):
    dst_deg = edge_index[1].reshape(NW, CHUNKSD, KD)
    src4 = edge_index[0].reshape(NW, HALVES, CH_H, K)
    dst4 = edge_index[1].reshape(NW, HALVES, CH_H, K)
    zeros16 = jnp.zeros((N_PAD, 16), jnp.float32)
    zerosD = jnp.zeros((N_PAD, D), jnp.float32)

    deg_parts = _sc_degree(dst_deg, zeros16)
    h = _tc_matmul(x, W)
    g = _tc_scale(h, deg_parts)
    parts = _sc_gather_scatter(g, src4, dst4, zerosD)
    return _tc_final(parts[0], parts[1], g, deg_parts, b.reshape(1, D))


# speedup vs baseline: 37.1852x; 1.0180x over previous
import functools

import jax
import jax.numpy as jnp
from jax import lax
from jax.experimental import pallas as pl
from jax.experimental.pallas import tpu as pltpu
from jax.experimental.pallas import tpu_sc as plsc

N = 10000
E = 320000
D = 128

NC = 2
NS = 16
NW = NC * NS
N_PAD = 10240
ROWS_PER_SUB = N_PAD // NS

DEG_REGS = (E // NW) // 16

K = 100
CHUNKS = (E // NW) // K
HALVES = 2
CH_H = CHUNKS // HALVES
NBUF = 2
SB = 10
SBLOCKS_H = CH_H // SB

_MESH = plsc.VectorSubcoreMesh(
    core_axis_name="c", subcore_axis_name="s", num_cores=NC, num_subcores=NS
)


@functools.partial(
    pl.kernel,
    out_type=jax.ShapeDtypeStruct((NW, N_PAD), jnp.float32),
    mesh=_MESH,
    scratch_types=[
        pltpu.VMEM((DEG_REGS, 16), jnp.int32),
        pltpu.VMEM((N_PAD,), jnp.float32),
    ],
    compiler_params=pltpu.CompilerParams(needs_layout_passes=False),
)
def _sc_degree(dst_hbm, out_hbm, idx_v, hist_v):
    cid = lax.axis_index("c")
    sid = lax.axis_index("s")
    wid = sid * NC + cid

    @pl.loop(0, N_PAD // 16)
    def _(j):
        hist_v[pl.ds(j * 16, 16)] = jnp.zeros((16,), jnp.float32)

    pltpu.sync_copy(dst_hbm.at[wid], idx_v)
    ones = jnp.full((16,), 1.0, jnp.float32)

    @pl.loop(0, DEG_REGS)
    def _(j):
        plsc.addupdate_scatter(hist_v, [idx_v[j, :]], ones)

    pltpu.sync_copy(hist_v, out_hbm.at[wid])


@functools.partial(
    pl.kernel,
    out_type=jax.ShapeDtypeStruct((NC, N_PAD, D), jnp.float32),
    mesh=_MESH,
    scratch_types=[
        pltpu.VMEM((CH_H, K), jnp.int32),
        pltpu.VMEM((CH_H, K), jnp.int32),
        pltpu.VMEM((NBUF, K, D), jnp.float32),
        pltpu.VMEM_SHARED((N_PAD, D), jnp.float32),
        pltpu.SemaphoreType.DMA((NBUF,)),
        pltpu.SemaphoreType.DMA((NBUF,)),
    ],
)
def _sc_gather_scatter(g_hbm, src_hbm, dst_hbm, zeros_hbm, out_hbm,
                       src_v, dst_v, rows_v, acc, gsem, ssem):
    cid = lax.axis_index("c")
    sid = lax.axis_index("s")
    wid = sid * NC + cid

    pltpu.sync_copy(
        zeros_hbm.at[pl.ds(sid * ROWS_PER_SUB, ROWS_PER_SUB)],
        acc.at[pl.ds(sid * ROWS_PER_SUB, ROWS_PER_SUB)],
    )
    plsc.subcore_barrier()

    for h in range(HALVES):
        pltpu.sync_copy(src_hbm.at[wid].at[h], src_v)
        pltpu.sync_copy(dst_hbm.at[wid].at[h], dst_v)

        @pl.loop(0, SBLOCKS_H)
        def _(sb):
            base = sb * SB
            gds = [None] * NBUF
            sds = [None] * NBUF
            gds[0] = pltpu.async_copy(g_hbm.at[src_v.at[base]],
                                      rows_v.at[0], gsem.at[0])
            for i in range(SB):
                b = i % 2
                nb = 1 - b
                if i + 1 < SB:
                    if i >= 1:
                        sds[nb].wait()
                    gds[nb] = pltpu.async_copy(
                        g_hbm.at[src_v.at[base + i + 1]], rows_v.at[nb],
                        gsem.at[nb])
                gds[b].wait()
                sds[b] = pltpu.async_copy(rows_v.at[b],
                                          acc.at[dst_v.at[base + i]],
                                          ssem.at[b], add=True)
            sds[0].wait()
            sds[1].wait()

    plsc.subcore_barrier()
    pltpu.sync_copy(
        acc.at[pl.ds(sid * ROWS_PER_SUB, ROWS_PER_SUB)],
        out_hbm.at[cid].at[pl.ds(sid * ROWS_PER_SUB, ROWS_PER_SUB)],
    )


_RB = 2000
_GRID = N // _RB


def _dis_from_parts(dp):
    deg = jnp.sum(dp, axis=1, keepdims=True) + 1.0
    return lax.rsqrt(deg)


def _matmul_body(x_ref, w_ref, h_ref):
    h_ref[...] = jnp.dot(x_ref[...], w_ref[...],
                         preferred_element_type=jnp.float32)


def _tc_matmul(x, W):
    return pl.pallas_call(
        _matmul_body,
        grid=(_GRID,),
        in_specs=[
            pl.BlockSpec((_RB, D), lambda i: (i, 0)),
            pl.BlockSpec((D, D), lambda i: (0, 0)),
        ],
        out_specs=pl.BlockSpec((_RB, D), lambda i: (i, 0)),
        out_shape=jax.ShapeDtypeStruct((N, D), jnp.float32),
    )(x, W)


def _scale_body(h_ref, dp_ref, g_ref):
    g_ref[...] = _dis_from_parts(dp_ref[...]) * h_ref[...]


def _tc_scale(h, deg_parts):
    return pl.pallas_call(
        _scale_body,
        grid=(_GRID,),
        in_specs=[
            pl.BlockSpec((_RB, D), lambda i: (i, 0)),
            pl.BlockSpec((_RB, NW), lambda i: (i, 0)),
        ],
        out_specs=pl.BlockSpec((_RB, D), lambda i: (i, 0)),
        out_shape=jax.ShapeDtypeStruct((N, D), jnp.float32),
    )(h, deg_parts)


def _final_body(p0_ref, p1_ref, g_ref, dp_ref, b_ref, o_ref):
    dis = _dis_from_parts(dp_ref[...])
    o = dis * (p0_ref[...] + p1_ref[...] + g_ref[...]) + b_ref[...]
    m = jnp.max(o, axis=1, keepdims=True)
    e = jnp.exp(o - m)
    z = jnp.sum(e, axis=1, keepdims=True)
    o_ref[...] = (o - m) - jnp.log(z)


def _tc_final(p0, p1, g, deg_parts, b):
    return pl.pallas_call(
        _final_body,
        grid=(_GRID,),
        in_specs=[
            pl.BlockSpec((_RB, D), lambda i: (i, 0)),
            pl.BlockSpec((_RB, D), lambda i: (i, 0)),
            pl.BlockSpec((_RB, D), lambda i: (i, 0)),
            pl.BlockSpec((_RB, NW), lambda i: (i, 0)),
            pl.BlockSpec((1, D), lambda i: (0, 0)),
        ],
        out_specs=pl.BlockSpec((_RB, D), lambda i: (i, 0)),
        out_shape=jax.ShapeDtypeStruct((N, D), jnp.float32),
    )(p0, p1, g, deg_parts, b)


@jax.jit
def kernel(x, edge_index, W, b):
    dst_deg = edge_index[1].reshape(NW, DEG_REGS, 16)
    src4 = edge_index[0].reshape(NW, HALVES, CH_H, K)
    dst4 = edge_index[1].reshape(NW, HALVES, CH_H, K)
    zerosD = jnp.zeros((N_PAD, D), jnp.float32)

    deg_parts = _sc_degree(dst_deg).T
    h = _tc_matmul(x, W)
    g = _tc_scale(h, deg_parts)
    parts = _sc_gather_scatter(g, src4, dst4, zerosD)
    return _tc_final(parts[0], parts[1], g, deg_parts, b.reshape(1, D))


# speedup vs baseline: 37.3367x; 1.0041x over previous
import functools

import jax
import jax.numpy as jnp
from jax import lax
from jax.experimental import pallas as pl
from jax.experimental.pallas import tpu as pltpu
from jax.experimental.pallas import tpu_sc as plsc

N = 10000
E = 320000
D = 128

NC = 2
NS = 16
NW = NC * NS
N_PAD = 10240
ROWS_PER_SUB = N_PAD // NS

DEG_REGS = (E // NW) // 16

K = 100
CHUNKS = (E // NW) // K
HALVES = 2
CH_H = CHUNKS // HALVES
NBUF = 2
SB = 10
SBLOCKS_H = CH_H // SB

_MESH = plsc.VectorSubcoreMesh(
    core_axis_name="c", subcore_axis_name="s", num_cores=NC, num_subcores=NS
)


@functools.partial(
    pl.kernel,
    out_type=jax.ShapeDtypeStruct((NW, N_PAD), jnp.float32),
    mesh=_MESH,
    scratch_types=[
        pltpu.VMEM((DEG_REGS, 16), jnp.int32),
        pltpu.VMEM((N_PAD,), jnp.float32),
    ],
    compiler_params=pltpu.CompilerParams(needs_layout_passes=False),
)
def _sc_degree(dst_hbm, out_hbm, idx_v, hist_v):
    cid = lax.axis_index("c")
    sid = lax.axis_index("s")
    wid = sid * NC + cid

    @pl.loop(0, N_PAD // 16)
    def _(j):
        hist_v[pl.ds(j * 16, 16)] = jnp.zeros((16,), jnp.float32)

    pltpu.sync_copy(dst_hbm.at[wid], idx_v)
    ones = jnp.full((16,), 1.0, jnp.float32)

    @pl.loop(0, DEG_REGS)
    def _(j):
        plsc.addupdate_scatter(hist_v, [idx_v[j, :]], ones)

    pltpu.sync_copy(hist_v, out_hbm.at[wid])


@functools.partial(
    pl.kernel,
    out_type=jax.ShapeDtypeStruct((NC, N_PAD, D), jnp.float32),
    mesh=_MESH,
    scratch_types=[
        pltpu.VMEM((CH_H, K), jnp.int32),
        pltpu.VMEM((CH_H, K), jnp.int32),
        pltpu.VMEM((NBUF, K, D), jnp.float32),
        pltpu.VMEM_SHARED((N_PAD, D), jnp.float32),
        pltpu.SemaphoreType.DMA((NBUF,)),
        pltpu.SemaphoreType.DMA((NBUF,)),
    ],
)
def _sc_gather_scatter(g_hbm, src_hbm, dst_hbm, zeros_hbm, out_hbm,
                       src_v, dst_v, rows_v, acc, gsem, ssem):
    cid = lax.axis_index("c")
    sid = lax.axis_index("s")
    wid = sid * NC + cid

    pltpu.sync_copy(
        zeros_hbm.at[pl.ds(sid * ROWS_PER_SUB, ROWS_PER_SUB)],
        acc.at[pl.ds(sid * ROWS_PER_SUB, ROWS_PER_SUB)],
    )
    plsc.subcore_barrier()

    for h in range(HALVES):
        pltpu.sync_copy(src_hbm.at[wid].at[h], src_v)
        pltpu.sync_copy(dst_hbm.at[wid].at[h], dst_v)

        @pl.loop(0, SBLOCKS_H)
        def _(sb):
            base = sb * SB
            gds = [None] * NBUF
            sds = [None] * NBUF
            gds[0] = pltpu.async_copy(g_hbm.at[src_v.at[base]],
                                      rows_v.at[0], gsem.at[0])
            for i in range(SB):
                b = i % 2
                nb = 1 - b
                if i + 1 < SB:
                    if i >= 1:
                        sds[nb].wait()
                    gds[nb] = pltpu.async_copy(
                        g_hbm.at[src_v.at[base + i + 1]], rows_v.at[nb],
                        gsem.at[nb])
                gds[b].wait()
                sds[b] = pltpu.async_copy(rows_v.at[b],
                                          acc.at[dst_v.at[base + i]],
                                          ssem.at[b], add=True)
            sds[0].wait()
            sds[1].wait()

    plsc.subcore_barrier()
    pltpu.sync_copy(
        acc.at[pl.ds(sid * ROWS_PER_SUB, ROWS_PER_SUB)],
        out_hbm.at[cid].at[pl.ds(sid * ROWS_PER_SUB, ROWS_PER_SUB)],
    )


_RB = 2000
_GRID = N // _RB


def _dis_from_parts(dp):
    deg = jnp.sum(dp, axis=1, keepdims=True) + 1.0
    return lax.rsqrt(deg)


def _scale_body(x_ref, w_ref, dp_ref, g_ref):
    h = jnp.dot(x_ref[...], w_ref[...], preferred_element_type=jnp.float32)
    g_ref[...] = _dis_from_parts(dp_ref[...]) * h


def _tc_scale(x, W, deg_parts):
    return pl.pallas_call(
        _scale_body,
        grid=(_GRID,),
        in_specs=[
            pl.BlockSpec((_RB, D), lambda i: (i, 0)),
            pl.BlockSpec((D, D), lambda i: (0, 0)),
            pl.BlockSpec((_RB, NW), lambda i: (i, 0)),
        ],
        out_specs=pl.BlockSpec((_RB, D), lambda i: (i, 0)),
        out_shape=jax.ShapeDtypeStruct((N, D), jnp.float32),
    )(x, W, deg_parts)


def _final_body(p0_ref, p1_ref, g_ref, dp_ref, b_ref, o_ref):
    dis = _dis_from_parts(dp_ref[...])
    o = dis * (p0_ref[...] + p1_ref[...] + g_ref[...]) + b_ref[...]
    m = jnp.max(o, axis=1, keepdims=True)
    e = jnp.exp(o - m)
    z = jnp.sum(e, axis=1, keepdims=True)
    o_ref[...] = (o - m) - jnp.log(z)


def _tc_final(p0, p1, g, deg_parts, b):
    return pl.pallas_call(
        _final_body,
        grid=(_GRID,),
        in_specs=[
            pl.BlockSpec((_RB, D), lambda i: (i, 0)),
            pl.BlockSpec((_RB, D), lambda i: (i, 0)),
            pl.BlockSpec((_RB, D), lambda i: (i, 0)),
            pl.BlockSpec((_RB, NW), lambda i: (i, 0)),
            pl.BlockSpec((1, D), lambda i: (0, 0)),
        ],
        out_specs=pl.BlockSpec((_RB, D), lambda i: (i, 0)),
        out_shape=jax.ShapeDtypeStruct((N, D), jnp.float32),
    )(p0, p1, g, deg_parts, b)


@jax.jit
def kernel(x, edge_index, W, b):
    dst_deg = edge_index[1].reshape(NW, DEG_REGS, 16)
    src4 = edge_index[0].reshape(NW, HALVES, CH_H, K)
    dst4 = edge_index[1].reshape(NW, HALVES, CH_H, K)
    zerosD = jnp.zeros((N_PAD, D), jnp.float32)

    deg_parts = _sc_degree(dst_deg).T
    g = _tc_scale(x, W, deg_parts)
    parts = _sc_gather_scatter(g, src4, dst4, zerosD)
    return _tc_final(parts[0], parts[1], g, deg_parts, b.reshape(1, D))
